# Initial kernel scaffold; baseline (speedup 1.0000x reference)
import jax
import jax.numpy as jnp
from jax.experimental import pallas as pl


def kernel(x, edge_index, embed, tmp, origin_pred, Wa, ba, Wb, bb, W1, W2, Wout):
    raise NotImplementedError("write your pallas kernel here")


# speedup vs baseline: 1.6669x; 1.6669x over previous
import functools
import jax
import jax.numpy as jnp
from jax.experimental import pallas as pl
from jax.experimental.pallas import tpu as pltpu

_INIT_K = 3
_TRAIN_TOPK = 10
_NC = 8


def _mm_body(a_ref, b_ref, o_ref, *, relu):
    acc = jnp.dot(a_ref[...], b_ref[...], preferred_element_type=jnp.float32)
    if relu:
        acc = jnp.maximum(acc, 0.0)
    o_ref[...] = acc


def _mm(a, b, relu=False, bm=512):
    m, k = a.shape
    n = b.shape[1]
    grid = (m // bm,)
    return pl.pallas_call(
        functools.partial(_mm_body, relu=relu),
        grid=grid,
        in_specs=[
            pl.BlockSpec((bm, k), lambda i: (i, 0)),
            pl.BlockSpec((k, n), lambda i: (0, 0)),
        ],
        out_specs=pl.BlockSpec((bm, n), lambda i: (i, 0)),
        out_shape=jax.ShapeDtypeStruct((m, n), jnp.float32),
    )(a, b)


def _edge_head_body(pg_ref, qg_ref, wb_ref, bab_ref, o_ref):
    a = jnp.maximum(pg_ref[...] + qg_ref[...] + bab_ref[...], 0.0)
    o_ref[...] = jax.lax.dot_general(
        a, wb_ref[...], (((1,), (0,)), ((), ())),
        preferred_element_type=jnp.float32)


def _edge_heads(pg, qg, wb_cat, ba_cat, bb_vec):
    e = pg.shape[0]
    be = 4096
    out = pl.pallas_call(
        _edge_head_body,
        grid=(e // be,),
        in_specs=[
            pl.BlockSpec((be, 192), lambda i: (i, 0)),
            pl.BlockSpec((be, 192), lambda i: (i, 0)),
            pl.BlockSpec((192, _INIT_K), lambda i: (0, 0)),
            pl.BlockSpec((1, 192), lambda i: (0, 0)),
        ],
        out_specs=pl.BlockSpec((be, _INIT_K), lambda i: (i, 0)),
        out_shape=jax.ShapeDtypeStruct((e, _INIT_K), jnp.float32),
    )(pg, qg, wb_cat, ba_cat)
    return out + bb_vec[None, :]


def _sample_body(g_ref, l_ref, beta_ref, o_ref):
    o_ref[...] = jax.nn.sigmoid((g_ref[...] + l_ref[...]) / beta_ref[0])


def _sample(gates, logits, beta):
    s, e = gates.shape
    be = 16384
    return pl.pallas_call(
        _sample_body,
        grid=(e // be,),
        in_specs=[
            pl.BlockSpec((s, be), lambda i: (0, i)),
            pl.BlockSpec((s, be), lambda i: (0, i)),
            pl.BlockSpec(memory_space=pltpu.SMEM),
        ],
        out_specs=pl.BlockSpec((s, be), lambda i: (0, i)),
        out_shape=jax.ShapeDtypeStruct((s, e), jnp.float32),
    )(gates, logits, beta)


def kernel(x, edge_index, embed, tmp, origin_pred, Wa, ba, Wb, bb, W1, W2, Wout):
    N = x.shape[0]
    E = edge_index.shape[1]
    e0 = edge_index[0].astype(jnp.int32)
    e1 = edge_index[1].astype(jnp.int32)
    beta = jnp.reshape(tmp, (1,))
    nkey = jax.random.key(1)

    gates = []
    for s in range(_INIT_K + 1):
        u = jax.random.uniform(jax.random.fold_in(nkey, s), (E,),
                               minval=1e-6, maxval=1.0 - 1e-6)
        gates.append(jnp.log(u) - jnp.log(1.0 - u))
    gates = jnp.stack(gates)

    WaL = jnp.concatenate([Wa[i, :512, :] for i in range(_INIT_K)], axis=1)
    WaR = jnp.concatenate([Wa[i, 512:, :] for i in range(_INIT_K)], axis=1)
    PQ = _mm(embed, jnp.concatenate([WaL, WaR], axis=1))
    P = PQ[:, :192]
    Q = PQ[:, 192:384]
    xW1 = _mm(x, W1, bm=512)

    pg = P[e0]
    qg = Q[e1]
    wb_cat = jnp.zeros((192, _INIT_K), jnp.float32)
    for i in range(_INIT_K):
        wb_cat = wb_cat.at[64 * i:64 * (i + 1), i].set(Wb[i][:, 0])
    ba_cat = jnp.concatenate([ba[i] for i in range(_INIT_K)]).reshape(1, 192)
    bb_vec = jnp.concatenate([bb[i] for i in range(_INIT_K)])
    e_h = _edge_heads(pg, qg, wb_cat, ba_cat, bb_vec)
    mask_logits = jnp.max(e_h, axis=1)

    logits_all = jnp.concatenate([mask_logits[None], e_h.T], axis=0)
    vals = _sample(gates, logits_all, beta)

    key = e0 * N + e1
    rkey = e1 * N + e0
    T = jnp.zeros((4, N * N), jnp.float32).at[:, key].add(vals)
    emask_all = (T[:, key] + T[:, rkey]) * 0.5
    edge_mask = emask_all[0]
    demask = emask_all[1:]

    def run_model(m):
        h = jnp.zeros((N, 128), jnp.float32).at[e1].add(xW1[e0] * m[:, None])
        h = jnp.maximum(h, 0.0)
        h2 = jnp.zeros((N, 128), jnp.float32).at[e1].add(h[e0] * m[:, None])
        h2 = _mm(h2, W2, relu=True)
        logits = (jnp.sum(h2, axis=0) / N) @ Wout
        return logits, jax.nn.softmax(logits)

    logits0, _ = run_model(edge_mask)
    res = jax.nn.softmax(logits0)

    k = round(_TRAIN_TOPK / 100.0 * E)

    def select(v):
        t = jax.lax.top_k(v, k)[0][-1]
        g = jnp.sum(v > t)
        eq = (v == t)
        tie = jnp.cumsum(eq.astype(jnp.int32)) - eq.astype(jnp.int32)
        return (v > t) | (eq & (g + tie < k))

    fminus, fplus, sel_last = [], [], None
    for i in range(_INIT_K):
        s = select(demask[i])
        sel_last = s
        _, p1 = run_model(jnp.where(s, 1.0, demask[i]))
        fminus.append(jnp.sum(jnp.abs(origin_pred - p1)))
        _, p2 = run_model(jnp.where(s, 1.0 - demask[i], 1.0))
        fplus.append(jnp.sum(jnp.abs(origin_pred - p2)))

    disen_max = jnp.max(demask, axis=0)
    pair = jnp.zeros((_NC,), jnp.float32)
    for i in range(_INIT_K):
        for j in range(i + 1, _INIT_K):
            mnt = jnp.where(sel_last, 1.0, jnp.maximum(demask[i], demask[j]))
            _, pp = run_model(mnt)
            pair = pair + pp

    return (res, demask, disen_max, jnp.stack(fminus), jnp.stack(fplus), pair)


# speedup vs baseline: 3.5325x; 2.1192x over previous
import functools
import jax
import jax.numpy as jnp
from jax import lax
from jax.experimental import pallas as pl
from jax.experimental.pallas import tpu as pltpu
from jax.experimental.pallas import tpu_sc as plsc

_INIT_K = 3
_TRAIN_TOPK = 10
_NC = 8

_NW = 32
_CHUNK = 128


def _seg_body(table, e0g, e1g, scale_g, zeros_hbm, out,
              idx0, idx1, scv, rows, stage, acc, sem):
    c = lax.axis_index("c")
    s = lax.axis_index("s")
    wid = s * 2 + c
    nchunk = e0g.shape[0] // _NW

    pltpu.sync_copy(zeros_hbm, stage)
    pltpu.sync_copy(stage, acc.at[pl.ds(s * 256, 256)])

    base = wid * nchunk
    pltpu.sync_copy(e0g.at[pl.ds(base, nchunk)], idx0)
    pltpu.sync_copy(e1g.at[pl.ds(base, nchunk)], idx1)
    pltpu.sync_copy(scale_g.at[pl.ds(base * _CHUNK, nchunk * _CHUNK)], scv)
    plsc.subcore_barrier()

    onehots = [jnp.float32(0) + jnp.where(
        jnp.arange(16) == l, jnp.float32(1), jnp.float32(0)) for l in range(16)]

    def chunk(j, carry):
        pltpu.async_copy(table.at[idx0.at[j]], rows, sem).wait()
        for g in range(_CHUNK // 16):
            grp = scv[pl.ds(j * _CHUNK + g * 16, 16)]
            for l in range(16):
                r = g * 16 + l
                b = jnp.sum(grp * onehots[l])
                for k in range(8):
                    rows[r, pl.ds(k * 16, 16)] = rows[r, pl.ds(k * 16, 16)] * b
        pltpu.sync_copy(rows, acc.at[idx1.at[j]], add=True)
        return carry

    lax.fori_loop(0, nchunk, chunk, 0)
    plsc.subcore_barrier()
    pltpu.sync_copy(acc.at[pl.ds(s * 256, 256)], stage)
    pltpu.sync_copy(stage, out.at[c, pl.ds(s * 256, 256)])


def _seg_pass(table, e0g, e1g, scale_g, zeros_hbm):
    n = table.shape[0]
    mesh = plsc.VectorSubcoreMesh(core_axis_name="c", subcore_axis_name="s")
    f = pl.kernel(
        _seg_body,
        out_type=jax.ShapeDtypeStruct((2, n, 128), jnp.float32),
        mesh=mesh,
        scratch_types=[
            pltpu.VMEM((e0g.shape[0] // _NW, _CHUNK), jnp.int32),
            pltpu.VMEM((e0g.shape[0] // _NW, _CHUNK), jnp.int32),
            pltpu.VMEM(((e0g.shape[0] // _NW) * _CHUNK,), jnp.float32),
            pltpu.VMEM((_CHUNK, 128), jnp.float32),
            pltpu.VMEM((256, 128), jnp.float32),
            pltpu.VMEM_SHARED((n, 128), jnp.float32),
            pltpu.SemaphoreType.DMA,
        ],
        compiler_params=pltpu.CompilerParams(needs_layout_passes=False),
    )
    return f(table, e0g, e1g, scale_g, zeros_hbm)


def _mm_body(a_ref, b_ref, o_ref, *, relu):
    acc = jnp.dot(a_ref[...], b_ref[...], preferred_element_type=jnp.float32)
    if relu:
        acc = jnp.maximum(acc, 0.0)
    o_ref[...] = acc


def _mm(a, b, relu=False, bm=512):
    m, k = a.shape
    n = b.shape[1]
    grid = (m // bm,)
    return pl.pallas_call(
        functools.partial(_mm_body, relu=relu),
        grid=grid,
        in_specs=[
            pl.BlockSpec((bm, k), lambda i: (i, 0)),
            pl.BlockSpec((k, n), lambda i: (0, 0)),
        ],
        out_specs=pl.BlockSpec((bm, n), lambda i: (i, 0)),
        out_shape=jax.ShapeDtypeStruct((m, n), jnp.float32),
    )(a, b)


def _edge_head_body(pg_ref, qg_ref, wb_ref, bab_ref, o_ref):
    a = jnp.maximum(pg_ref[...] + qg_ref[...] + bab_ref[...], 0.0)
    o_ref[...] = jax.lax.dot_general(
        a, wb_ref[...], (((1,), (0,)), ((), ())),
        preferred_element_type=jnp.float32)


def _edge_heads(pg, qg, wb_cat, ba_cat, bb_vec):
    e = pg.shape[0]
    be = 4096
    out = pl.pallas_call(
        _edge_head_body,
        grid=(e // be,),
        in_specs=[
            pl.BlockSpec((be, 192), lambda i: (i, 0)),
            pl.BlockSpec((be, 192), lambda i: (i, 0)),
            pl.BlockSpec((192, _INIT_K), lambda i: (0, 0)),
            pl.BlockSpec((1, 192), lambda i: (0, 0)),
        ],
        out_specs=pl.BlockSpec((be, _INIT_K), lambda i: (i, 0)),
        out_shape=jax.ShapeDtypeStruct((e, _INIT_K), jnp.float32),
    )(pg, qg, wb_cat, ba_cat)
    return out + bb_vec[None, :]


def _sample_body(g_ref, l_ref, beta_ref, o_ref):
    o_ref[...] = jax.nn.sigmoid((g_ref[...] + l_ref[...]) / beta_ref[0])


def _sample(gates, logits, beta):
    s, e = gates.shape
    be = 16384
    return pl.pallas_call(
        _sample_body,
        grid=(e // be,),
        in_specs=[
            pl.BlockSpec((s, be), lambda i: (0, i)),
            pl.BlockSpec((s, be), lambda i: (0, i)),
            pl.BlockSpec(memory_space=pltpu.SMEM),
        ],
        out_specs=pl.BlockSpec((s, be), lambda i: (0, i)),
        out_shape=jax.ShapeDtypeStruct((s, e), jnp.float32),
    )(gates, logits, beta)


def kernel(x, edge_index, embed, tmp, origin_pred, Wa, ba, Wb, bb, W1, W2, Wout):
    N = x.shape[0]
    E = edge_index.shape[1]
    e0 = edge_index[0].astype(jnp.int32)
    e1 = edge_index[1].astype(jnp.int32)
    beta = jnp.reshape(tmp, (1,))
    nkey = jax.random.key(1)

    gates = []
    for s in range(_INIT_K + 1):
        u = jax.random.uniform(jax.random.fold_in(nkey, s), (E,),
                               minval=1e-6, maxval=1.0 - 1e-6)
        gates.append(jnp.log(u) - jnp.log(1.0 - u))
    gates = jnp.stack(gates)

    WaL = jnp.concatenate([Wa[i, :512, :] for i in range(_INIT_K)], axis=1)
    WaR = jnp.concatenate([Wa[i, 512:, :] for i in range(_INIT_K)], axis=1)
    PQ = _mm(embed, jnp.concatenate([WaL, WaR], axis=1))
    P = PQ[:, :192]
    Q = PQ[:, 192:384]
    xW1 = _mm(x, W1, bm=512)

    pg = P[e0]
    qg = Q[e1]
    wb_cat = jnp.zeros((192, _INIT_K), jnp.float32)
    for i in range(_INIT_K):
        wb_cat = wb_cat.at[64 * i:64 * (i + 1), i].set(Wb[i][:, 0])
    ba_cat = jnp.concatenate([ba[i] for i in range(_INIT_K)]).reshape(1, 192)
    bb_vec = jnp.concatenate([bb[i] for i in range(_INIT_K)])
    e_h = _edge_heads(pg, qg, wb_cat, ba_cat, bb_vec)
    mask_logits = jnp.max(e_h, axis=1)

    logits_all = jnp.concatenate([mask_logits[None], e_h.T], axis=0)
    vals = _sample(gates, logits_all, beta)

    key = e0 * N + e1
    rkey = e1 * N + e0
    T = jnp.zeros((4, N * N), jnp.float32).at[:, key].add(vals)
    emask_all = (T[:, key] + T[:, rkey]) * 0.5
    edge_mask = emask_all[0]
    demask = emask_all[1:]

    e0g = e0.reshape(E // _CHUNK, _CHUNK)
    e1g = e1.reshape(E // _CHUNK, _CHUNK)
    z256 = jnp.zeros((256, 128), jnp.float32)

    def run_model(m):
        p = _seg_pass(xW1, e0g, e1g, m, z256)
        h = jnp.maximum(p[0] + p[1], 0.0)
        p2 = _seg_pass(h, e0g, e1g, m, z256)
        h2 = _mm(p2[0] + p2[1], W2, relu=True)
        logits = (jnp.sum(h2, axis=0) / N) @ Wout
        return logits, jax.nn.softmax(logits)

    logits0, _ = run_model(edge_mask)
    res = jax.nn.softmax(logits0)

    k = round(_TRAIN_TOPK / 100.0 * E)

    def select(v):
        t = jax.lax.top_k(v, k)[0][-1]
        g = jnp.sum(v > t)
        eq = (v == t)
        tie = jnp.cumsum(eq.astype(jnp.int32)) - eq.astype(jnp.int32)
        return (v > t) | (eq & (g + tie < k))

    fminus, fplus, sel_last = [], [], None
    for i in range(_INIT_K):
        s = select(demask[i])
        sel_last = s
        _, p1 = run_model(jnp.where(s, 1.0, demask[i]))
        fminus.append(jnp.sum(jnp.abs(origin_pred - p1)))
        _, p2 = run_model(jnp.where(s, 1.0 - demask[i], 1.0))
        fplus.append(jnp.sum(jnp.abs(origin_pred - p2)))

    disen_max = jnp.max(demask, axis=0)
    pair = jnp.zeros((_NC,), jnp.float32)
    for i in range(_INIT_K):
        for j in range(i + 1, _INIT_K):
            mnt = jnp.where(sel_last, 1.0, jnp.maximum(demask[i], demask[j]))
            _, pp = run_model(mnt)
            pair = pair + pp

    return (res, demask, disen_max, jnp.stack(fminus), jnp.stack(fplus), pair)


# speedup vs baseline: 3.6178x; 1.0241x over previous
import functools
import jax
import jax.numpy as jnp
from jax import lax
from jax.experimental import pallas as pl
from jax.experimental.pallas import tpu as pltpu
from jax.experimental.pallas import tpu_sc as plsc

_INIT_K = 3
_TRAIN_TOPK = 10
_NC = 8

_NW = 32
_CHUNK = 128


def _seg_body(table, e0g, e1g, scale_g, zeros_hbm, out,
              idx0, idx1, scv, rows, rows2, stage, acc, sem, sem2):
    c = lax.axis_index("c")
    s = lax.axis_index("s")
    wid = s * 2 + c
    nchunk = e0g.shape[0] // _NW

    pltpu.sync_copy(zeros_hbm, stage)
    pltpu.sync_copy(stage, acc.at[pl.ds(s * 256, 256)])

    base = wid * nchunk
    pltpu.sync_copy(e0g.at[pl.ds(base, nchunk)], idx0)
    pltpu.sync_copy(e1g.at[pl.ds(base, nchunk)], idx1)
    pltpu.sync_copy(scale_g.at[pl.ds(base * _CHUNK, nchunk * _CHUNK)], scv)
    plsc.subcore_barrier()

    lanes = [jnp.full((16, 1), l, jnp.int32) for l in range(16)]
    dnums = lax.GatherDimensionNumbers(
        offset_dims=(), collapsed_slice_dims=(0,), start_index_map=(0,))

    def process(j, rows_b, sem_b):
        pltpu.make_async_copy(table.at[idx0.at[j]], rows_b, sem_b).wait()
        for g in range(_CHUNK // 16):
            grp = scv[pl.ds(j * _CHUNK + g * 16, 16)]
            for l in range(16):
                r = g * 16 + l
                b = lax.gather(grp, lanes[l], dnums, (1,),
                               mode=lax.GatherScatterMode.PROMISE_IN_BOUNDS)
                for k in range(8):
                    rows_b[r, pl.ds(k * 16, 16)] = rows_b[r, pl.ds(k * 16, 16)] * b
        pltpu.sync_copy(rows_b, acc.at[idx1.at[j]], add=True)

        @pl.when(j + 2 < nchunk)
        def _():
            pltpu.async_copy(table.at[idx0.at[j + 2]], rows_b, sem_b)

    pltpu.async_copy(table.at[idx0.at[0]], rows, sem)
    pltpu.async_copy(table.at[idx0.at[1]], rows2, sem2)

    def chunk(i, carry):
        process(i * 2, rows, sem)
        process(i * 2 + 1, rows2, sem2)
        return carry

    lax.fori_loop(0, nchunk // 2, chunk, 0)
    plsc.subcore_barrier()
    pltpu.sync_copy(acc.at[pl.ds(s * 256, 256)], stage)
    pltpu.sync_copy(stage, out.at[c, pl.ds(s * 256, 256)])


def _seg_pass(table, e0g, e1g, scale_g, zeros_hbm):
    n = table.shape[0]
    mesh = plsc.VectorSubcoreMesh(core_axis_name="c", subcore_axis_name="s")
    f = pl.kernel(
        _seg_body,
        out_type=jax.ShapeDtypeStruct((2, n, 128), jnp.float32),
        mesh=mesh,
        scratch_types=[
            pltpu.VMEM((e0g.shape[0] // _NW, _CHUNK), jnp.int32),
            pltpu.VMEM((e0g.shape[0] // _NW, _CHUNK), jnp.int32),
            pltpu.VMEM(((e0g.shape[0] // _NW) * _CHUNK,), jnp.float32),
            pltpu.VMEM((_CHUNK, 128), jnp.float32),
            pltpu.VMEM((_CHUNK, 128), jnp.float32),
            pltpu.VMEM((256, 128), jnp.float32),
            pltpu.VMEM_SHARED((n, 128), jnp.float32),
            pltpu.SemaphoreType.DMA,
            pltpu.SemaphoreType.DMA,
        ],
        compiler_params=pltpu.CompilerParams(needs_layout_passes=False),
    )
    return f(table, e0g, e1g, scale_g, zeros_hbm)


def _mm_body(a_ref, b_ref, o_ref, *, relu):
    acc = jnp.dot(a_ref[...], b_ref[...], preferred_element_type=jnp.float32)
    if relu:
        acc = jnp.maximum(acc, 0.0)
    o_ref[...] = acc


def _mm(a, b, relu=False, bm=512):
    m, k = a.shape
    n = b.shape[1]
    grid = (m // bm,)
    return pl.pallas_call(
        functools.partial(_mm_body, relu=relu),
        grid=grid,
        in_specs=[
            pl.BlockSpec((bm, k), lambda i: (i, 0)),
            pl.BlockSpec((k, n), lambda i: (0, 0)),
        ],
        out_specs=pl.BlockSpec((bm, n), lambda i: (i, 0)),
        out_shape=jax.ShapeDtypeStruct((m, n), jnp.float32),
    )(a, b)


def _edge_head_body(pg_ref, qg_ref, wb_ref, bab_ref, o_ref):
    a = jnp.maximum(pg_ref[...] + qg_ref[...] + bab_ref[...], 0.0)
    o_ref[...] = jax.lax.dot_general(
        a, wb_ref[...], (((1,), (0,)), ((), ())),
        preferred_element_type=jnp.float32)


def _edge_heads(pg, qg, wb_cat, ba_cat, bb_vec):
    e = pg.shape[0]
    be = 4096
    out = pl.pallas_call(
        _edge_head_body,
        grid=(e // be,),
        in_specs=[
            pl.BlockSpec((be, 192), lambda i: (i, 0)),
            pl.BlockSpec((be, 192), lambda i: (i, 0)),
            pl.BlockSpec((192, _INIT_K), lambda i: (0, 0)),
            pl.BlockSpec((1, 192), lambda i: (0, 0)),
        ],
        out_specs=pl.BlockSpec((be, _INIT_K), lambda i: (i, 0)),
        out_shape=jax.ShapeDtypeStruct((e, _INIT_K), jnp.float32),
    )(pg, qg, wb_cat, ba_cat)
    return out + bb_vec[None, :]


def _sample_body(g_ref, l_ref, beta_ref, o_ref):
    o_ref[...] = jax.nn.sigmoid((g_ref[...] + l_ref[...]) / beta_ref[0])


def _sample(gates, logits, beta):
    s, e = gates.shape
    be = 16384
    return pl.pallas_call(
        _sample_body,
        grid=(e // be,),
        in_specs=[
            pl.BlockSpec((s, be), lambda i: (0, i)),
            pl.BlockSpec((s, be), lambda i: (0, i)),
            pl.BlockSpec(memory_space=pltpu.SMEM),
        ],
        out_specs=pl.BlockSpec((s, be), lambda i: (0, i)),
        out_shape=jax.ShapeDtypeStruct((s, e), jnp.float32),
    )(gates, logits, beta)


def kernel(x, edge_index, embed, tmp, origin_pred, Wa, ba, Wb, bb, W1, W2, Wout):
    N = x.shape[0]
    E = edge_index.shape[1]
    e0 = edge_index[0].astype(jnp.int32)
    e1 = edge_index[1].astype(jnp.int32)
    beta = jnp.reshape(tmp, (1,))
    nkey = jax.random.key(1)

    gates = []
    for s in range(_INIT_K + 1):
        u = jax.random.uniform(jax.random.fold_in(nkey, s), (E,),
                               minval=1e-6, maxval=1.0 - 1e-6)
        gates.append(jnp.log(u) - jnp.log(1.0 - u))
    gates = jnp.stack(gates)

    WaL = jnp.concatenate([Wa[i, :512, :] for i in range(_INIT_K)], axis=1)
    WaR = jnp.concatenate([Wa[i, 512:, :] for i in range(_INIT_K)], axis=1)
    PQ = _mm(embed, jnp.concatenate([WaL, WaR], axis=1))
    P = PQ[:, :192]
    Q = PQ[:, 192:384]
    xW1 = _mm(x, W1, bm=512)

    pg = P[e0]
    qg = Q[e1]
    wb_cat = jnp.zeros((192, _INIT_K), jnp.float32)
    for i in range(_INIT_K):
        wb_cat = wb_cat.at[64 * i:64 * (i + 1), i].set(Wb[i][:, 0])
    ba_cat = jnp.concatenate([ba[i] for i in range(_INIT_K)]).reshape(1, 192)
    bb_vec = jnp.concatenate([bb[i] for i in range(_INIT_K)])
    e_h = _edge_heads(pg, qg, wb_cat, ba_cat, bb_vec)
    mask_logits = jnp.max(e_h, axis=1)

    logits_all = jnp.concatenate([mask_logits[None], e_h.T], axis=0)
    vals = _sample(gates, logits_all, beta)

    key = e0 * N + e1
    rkey = e1 * N + e0
    T = jnp.zeros((4, N * N), jnp.float32).at[:, key].add(vals)
    emask_all = (T[:, key] + T[:, rkey]) * 0.5
    edge_mask = emask_all[0]
    demask = emask_all[1:]

    e0g = e0.reshape(E // _CHUNK, _CHUNK)
    e1g = e1.reshape(E // _CHUNK, _CHUNK)
    z256 = jnp.zeros((256, 128), jnp.float32)

    def run_model(m):
        p = _seg_pass(xW1, e0g, e1g, m, z256)
        h = jnp.maximum(p[0] + p[1], 0.0)
        p2 = _seg_pass(h, e0g, e1g, m, z256)
        h2 = _mm(p2[0] + p2[1], W2, relu=True)
        logits = (jnp.sum(h2, axis=0) / N) @ Wout
        return logits, jax.nn.softmax(logits)

    logits0, _ = run_model(edge_mask)
    res = jax.nn.softmax(logits0)

    k = round(_TRAIN_TOPK / 100.0 * E)

    def select(v):
        t = jax.lax.top_k(v, k)[0][-1]
        g = jnp.sum(v > t)
        eq = (v == t)
        tie = jnp.cumsum(eq.astype(jnp.int32)) - eq.astype(jnp.int32)
        return (v > t) | (eq & (g + tie < k))

    fminus, fplus, sel_last = [], [], None
    for i in range(_INIT_K):
        s = select(demask[i])
        sel_last = s
        _, p1 = run_model(jnp.where(s, 1.0, demask[i]))
        fminus.append(jnp.sum(jnp.abs(origin_pred - p1)))
        _, p2 = run_model(jnp.where(s, 1.0 - demask[i], 1.0))
        fplus.append(jnp.sum(jnp.abs(origin_pred - p2)))

    disen_max = jnp.max(demask, axis=0)
    pair = jnp.zeros((_NC,), jnp.float32)
    for i in range(_INIT_K):
        for j in range(i + 1, _INIT_K):
            mnt = jnp.where(sel_last, 1.0, jnp.maximum(demask[i], demask[j]))
            _, pp = run_model(mnt)
            pair = pair + pp

    return (res, demask, disen_max, jnp.stack(fminus), jnp.stack(fplus), pair)


# speedup vs baseline: 3.8517x; 1.0647x over previous
import functools
import jax
import jax.numpy as jnp
from jax import lax
from jax.experimental import pallas as pl
from jax.experimental.pallas import tpu as pltpu
from jax.experimental.pallas import tpu_sc as plsc

_INIT_K = 3
_TRAIN_TOPK = 10
_NC = 8

_NW = 32
_CHUNK = 128


def _seg_body(table, e0g, e1g, scale_g, zeros_hbm, out,
              idx0, idx1, scv, rows, rows2, stage, acc, sem, sem2):
    c = lax.axis_index("c")
    s = lax.axis_index("s")
    wid = s * 2 + c
    nchunk = e0g.shape[0] // _NW

    pltpu.sync_copy(zeros_hbm, stage)
    pltpu.sync_copy(stage, acc.at[pl.ds(s * 256, 256)])

    base = wid * nchunk
    pltpu.sync_copy(e0g.at[pl.ds(base, nchunk)], idx0)
    pltpu.sync_copy(e1g.at[pl.ds(base, nchunk)], idx1)
    pltpu.sync_copy(scale_g.at[pl.ds(base * _CHUNK, nchunk * _CHUNK)], scv)
    plsc.subcore_barrier()

    lanes = [jnp.full((16, 1), l, jnp.int32) for l in range(16)]
    dnums = lax.GatherDimensionNumbers(
        offset_dims=(), collapsed_slice_dims=(0,), start_index_map=(0,))

    def process(j, rows_b, sem_b):
        pltpu.make_async_copy(table.at[idx0.at[j]], rows_b, sem_b).wait()
        for g in range(_CHUNK // 16):
            grp = scv[pl.ds(j * _CHUNK + g * 16, 16)]
            for l in range(16):
                r = g * 16 + l
                b = lax.gather(grp, lanes[l], dnums, (1,),
                               mode=lax.GatherScatterMode.PROMISE_IN_BOUNDS)
                for k in range(8):
                    rows_b[r, pl.ds(k * 16, 16)] = rows_b[r, pl.ds(k * 16, 16)] * b
        pltpu.sync_copy(rows_b, acc.at[idx1.at[j]], add=True)

        @pl.when(j + 2 < nchunk)
        def _():
            pltpu.async_copy(table.at[idx0.at[j + 2]], rows_b, sem_b)

    pltpu.async_copy(table.at[idx0.at[0]], rows, sem)
    pltpu.async_copy(table.at[idx0.at[1]], rows2, sem2)

    def chunk(i, carry):
        process(i * 2, rows, sem)
        process(i * 2 + 1, rows2, sem2)
        return carry

    lax.fori_loop(0, nchunk // 2, chunk, 0)
    plsc.subcore_barrier()
    pltpu.sync_copy(acc.at[pl.ds(s * 256, 256)], stage)
    pltpu.sync_copy(stage, out.at[c, pl.ds(s * 256, 256)])


def _seg_pass(table, e0g, e1g, scale_g, zeros_hbm):
    n = table.shape[0]
    mesh = plsc.VectorSubcoreMesh(core_axis_name="c", subcore_axis_name="s")
    f = pl.kernel(
        _seg_body,
        out_type=jax.ShapeDtypeStruct((2, n, 128), jnp.float32),
        mesh=mesh,
        scratch_types=[
            pltpu.VMEM((e0g.shape[0] // _NW, _CHUNK), jnp.int32),
            pltpu.VMEM((e0g.shape[0] // _NW, _CHUNK), jnp.int32),
            pltpu.VMEM(((e0g.shape[0] // _NW) * _CHUNK,), jnp.float32),
            pltpu.VMEM((_CHUNK, 128), jnp.float32),
            pltpu.VMEM((_CHUNK, 128), jnp.float32),
            pltpu.VMEM((256, 128), jnp.float32),
            pltpu.VMEM_SHARED((n, 128), jnp.float32),
            pltpu.SemaphoreType.DMA,
            pltpu.SemaphoreType.DMA,
        ],
        compiler_params=pltpu.CompilerParams(needs_layout_passes=False),
    )
    return f(table, e0g, e1g, scale_g, zeros_hbm)


def _mm_body(a_ref, b_ref, o_ref, *, relu):
    acc = jnp.dot(a_ref[...], b_ref[...], preferred_element_type=jnp.float32)
    if relu:
        acc = jnp.maximum(acc, 0.0)
    o_ref[...] = acc


def _mm(a, b, relu=False, bm=512):
    m, k = a.shape
    n = b.shape[1]
    grid = (m // bm,)
    return pl.pallas_call(
        functools.partial(_mm_body, relu=relu),
        grid=grid,
        in_specs=[
            pl.BlockSpec((bm, k), lambda i: (i, 0)),
            pl.BlockSpec((k, n), lambda i: (0, 0)),
        ],
        out_specs=pl.BlockSpec((bm, n), lambda i: (i, 0)),
        out_shape=jax.ShapeDtypeStruct((m, n), jnp.float32),
    )(a, b)


def _edge_head_body(pg_ref, qg_ref, wb_ref, bab_ref, o_ref):
    a = jnp.maximum(pg_ref[...] + qg_ref[...] + bab_ref[...], 0.0)
    o_ref[...] = jax.lax.dot_general(
        a, wb_ref[...], (((1,), (0,)), ((), ())),
        preferred_element_type=jnp.float32)


def _edge_heads(pg, qg, wb_cat, ba_cat, bb_vec):
    e = pg.shape[0]
    be = 4096
    out = pl.pallas_call(
        _edge_head_body,
        grid=(e // be,),
        in_specs=[
            pl.BlockSpec((be, 192), lambda i: (i, 0)),
            pl.BlockSpec((be, 192), lambda i: (i, 0)),
            pl.BlockSpec((192, _INIT_K), lambda i: (0, 0)),
            pl.BlockSpec((1, 192), lambda i: (0, 0)),
        ],
        out_specs=pl.BlockSpec((be, _INIT_K), lambda i: (i, 0)),
        out_shape=jax.ShapeDtypeStruct((e, _INIT_K), jnp.float32),
    )(pg, qg, wb_cat, ba_cat)
    return out + bb_vec[None, :]


def _sample_body(g_ref, l_ref, beta_ref, o_ref):
    o_ref[...] = jax.nn.sigmoid((g_ref[...] + l_ref[...]) / beta_ref[0])


def _sample(gates, logits, beta):
    s, e = gates.shape
    be = 16384
    return pl.pallas_call(
        _sample_body,
        grid=(e // be,),
        in_specs=[
            pl.BlockSpec((s, be), lambda i: (0, i)),
            pl.BlockSpec((s, be), lambda i: (0, i)),
            pl.BlockSpec(memory_space=pltpu.SMEM),
        ],
        out_specs=pl.BlockSpec((s, be), lambda i: (0, i)),
        out_shape=jax.ShapeDtypeStruct((s, e), jnp.float32),
    )(gates, logits, beta)


def kernel(x, edge_index, embed, tmp, origin_pred, Wa, ba, Wb, bb, W1, W2, Wout):
    N = x.shape[0]
    E = edge_index.shape[1]
    e0 = edge_index[0].astype(jnp.int32)
    e1 = edge_index[1].astype(jnp.int32)
    beta = jnp.reshape(tmp, (1,))
    nkey = jax.random.key(1)

    gates = []
    for s in range(_INIT_K + 1):
        u = jax.random.uniform(jax.random.fold_in(nkey, s), (E,),
                               minval=1e-6, maxval=1.0 - 1e-6)
        gates.append(jnp.log(u) - jnp.log(1.0 - u))
    gates = jnp.stack(gates)

    WaL = jnp.concatenate([Wa[i, :512, :] for i in range(_INIT_K)], axis=1)
    WaR = jnp.concatenate([Wa[i, 512:, :] for i in range(_INIT_K)], axis=1)
    PQ = _mm(embed, jnp.concatenate([WaL, WaR], axis=1))
    P = PQ[:, :192]
    Q = PQ[:, 192:384]
    xW1 = _mm(x, W1, bm=512)

    pg = P[e0]
    qg = Q[e1]
    wb_cat = jnp.zeros((192, _INIT_K), jnp.float32)
    for i in range(_INIT_K):
        wb_cat = wb_cat.at[64 * i:64 * (i + 1), i].set(Wb[i][:, 0])
    ba_cat = jnp.concatenate([ba[i] for i in range(_INIT_K)]).reshape(1, 192)
    bb_vec = jnp.concatenate([bb[i] for i in range(_INIT_K)])
    e_h = _edge_heads(pg, qg, wb_cat, ba_cat, bb_vec)
    mask_logits = jnp.max(e_h, axis=1)

    logits_all = jnp.concatenate([mask_logits[None], e_h.T], axis=0)
    vals = _sample(gates, logits_all, beta)

    key = e0 * N + e1
    rkey = e1 * N + e0
    T = jnp.zeros((N * N, 4), jnp.float32).at[key].add(vals.T)
    emask_all = (T[key] + T[rkey]).T * 0.5
    edge_mask = emask_all[0]
    demask = emask_all[1:]

    e0g = e0.reshape(E // _CHUNK, _CHUNK)
    e1g = e1.reshape(E // _CHUNK, _CHUNK)
    z256 = jnp.zeros((256, 128), jnp.float32)

    def run_model(m):
        p = _seg_pass(xW1, e0g, e1g, m, z256)
        h = jnp.maximum(p[0] + p[1], 0.0)
        p2 = _seg_pass(h, e0g, e1g, m, z256)
        h2 = _mm(p2[0] + p2[1], W2, relu=True)
        logits = (jnp.sum(h2, axis=0) / N) @ Wout
        return logits, jax.nn.softmax(logits)

    logits0, _ = run_model(edge_mask)
    res = jax.nn.softmax(logits0)

    k = round(_TRAIN_TOPK / 100.0 * E)

    def kth_largest(v):
        bits = lax.bitcast_convert_type(v, jnp.int32)
        lo = jnp.zeros((3,), jnp.int32)
        hi = jnp.full((3,), 0x7f800000, jnp.int32)

        def body(_, lohi):
            lo, hi = lohi
            mid = lo + (hi - lo + 1) // 2
            cnt = jnp.sum(bits >= mid[:, None], axis=1)
            ok = cnt >= k
            return jnp.where(ok, mid, lo), jnp.where(ok, hi, mid - 1)

        lo, hi = lax.fori_loop(0, 31, body, (lo, hi))
        return lax.bitcast_convert_type(lo, jnp.float32)

    tks = kth_largest(demask)

    def select(v, t):
        g = jnp.sum(v > t)
        eq = (v == t)
        tie = jnp.cumsum(eq.astype(jnp.int32)) - eq.astype(jnp.int32)
        return (v > t) | (eq & (g + tie < k))

    fminus, fplus, sel_last = [], [], None
    for i in range(_INIT_K):
        s = select(demask[i], tks[i])
        sel_last = s
        _, p1 = run_model(jnp.where(s, 1.0, demask[i]))
        fminus.append(jnp.sum(jnp.abs(origin_pred - p1)))
        _, p2 = run_model(jnp.where(s, 1.0 - demask[i], 1.0))
        fplus.append(jnp.sum(jnp.abs(origin_pred - p2)))

    disen_max = jnp.max(demask, axis=0)
    pair = jnp.zeros((_NC,), jnp.float32)
    for i in range(_INIT_K):
        for j in range(i + 1, _INIT_K):
            mnt = jnp.where(sel_last, 1.0, jnp.maximum(demask[i], demask[j]))
            _, pp = run_model(mnt)
            pair = pair + pp

    return (res, demask, disen_max, jnp.stack(fminus), jnp.stack(fplus), pair)


# speedup vs baseline: 3.9779x; 1.0328x over previous
import functools
import jax
import jax.numpy as jnp
from jax import lax
from jax.experimental import pallas as pl
from jax.experimental.pallas import tpu as pltpu
from jax.experimental.pallas import tpu_sc as plsc

_INIT_K = 3
_TRAIN_TOPK = 10
_NC = 8

_NW = 32
_CHUNK = 128


def _seg_body(table, e0g, e1g, scale_g, zeros_hbm, out,
              idx0, idx1, scv, rows, rows2, stage, acc, sem, sem2,
              sem_s1, sem_s2):
    c = lax.axis_index("c")
    s = lax.axis_index("s")
    wid = s * 2 + c
    nchunk = e0g.shape[0] // _NW

    pltpu.sync_copy(zeros_hbm, stage)
    pltpu.sync_copy(stage, acc.at[pl.ds(s * 256, 256)])

    base = wid * nchunk
    pltpu.sync_copy(e0g.at[pl.ds(base, nchunk)], idx0)
    pltpu.sync_copy(e1g.at[pl.ds(base, nchunk)], idx1)
    pltpu.sync_copy(scale_g.at[pl.ds(base * _CHUNK, nchunk * _CHUNK)], scv)
    plsc.subcore_barrier()

    lanes = [jnp.full((16, 1), l, jnp.int32) for l in range(16)]
    dnums = lax.GatherDimensionNumbers(
        offset_dims=(), collapsed_slice_dims=(0,), start_index_map=(0,))

    def process(j, rows_b, sem_b, sem_s, o_rows, o_gsem, o_ssem):
        pltpu.make_async_copy(table.at[idx0.at[j]], rows_b, sem_b).wait()
        @pl.when((j >= 1) & (j + 1 < nchunk))
        def _():
            pltpu.make_async_copy(o_rows, acc.at[idx1.at[j - 1]], o_ssem).wait()
            pltpu.async_copy(table.at[idx0.at[j + 1]], o_rows, o_gsem)
        for g in range(_CHUNK // 16):
            grp = scv[pl.ds(j * _CHUNK + g * 16, 16)]
            for l in range(16):
                r = g * 16 + l
                b = lax.gather(grp, lanes[l], dnums, (1,),
                               mode=lax.GatherScatterMode.PROMISE_IN_BOUNDS)
                for k in range(8):
                    rows_b[r, pl.ds(k * 16, 16)] = rows_b[r, pl.ds(k * 16, 16)] * b
        pltpu.async_copy(rows_b, acc.at[idx1.at[j]], sem_s, add=True)

    pltpu.async_copy(table.at[idx0.at[0]], rows, sem)
    pltpu.async_copy(table.at[idx0.at[1]], rows2, sem2)

    def chunk(i, carry):
        process(i * 2, rows, sem, sem_s1, rows2, sem2, sem_s2)
        process(i * 2 + 1, rows2, sem2, sem_s2, rows, sem, sem_s1)
        return carry

    lax.fori_loop(0, nchunk // 2, chunk, 0)
    pltpu.make_async_copy(rows, acc.at[idx1.at[nchunk - 2]], sem_s1).wait()
    pltpu.make_async_copy(rows2, acc.at[idx1.at[nchunk - 1]], sem_s2).wait()
    plsc.subcore_barrier()
    pltpu.sync_copy(acc.at[pl.ds(s * 256, 256)], stage)
    pltpu.sync_copy(stage, out.at[c, pl.ds(s * 256, 256)])


def _seg_pass(table, e0g, e1g, scale_g, zeros_hbm):
    n = table.shape[0]
    mesh = plsc.VectorSubcoreMesh(core_axis_name="c", subcore_axis_name="s")
    f = pl.kernel(
        _seg_body,
        out_type=jax.ShapeDtypeStruct((2, n, 128), jnp.float32),
        mesh=mesh,
        scratch_types=[
            pltpu.VMEM((e0g.shape[0] // _NW, _CHUNK), jnp.int32),
            pltpu.VMEM((e0g.shape[0] // _NW, _CHUNK), jnp.int32),
            pltpu.VMEM(((e0g.shape[0] // _NW) * _CHUNK,), jnp.float32),
            pltpu.VMEM((_CHUNK, 128), jnp.float32),
            pltpu.VMEM((_CHUNK, 128), jnp.float32),
            pltpu.VMEM((256, 128), jnp.float32),
            pltpu.VMEM_SHARED((n, 128), jnp.float32),
            pltpu.SemaphoreType.DMA,
            pltpu.SemaphoreType.DMA,
            pltpu.SemaphoreType.DMA,
            pltpu.SemaphoreType.DMA,
        ],
        compiler_params=pltpu.CompilerParams(needs_layout_passes=False),
    )
    return f(table, e0g, e1g, scale_g, zeros_hbm)


def _mm_body(a_ref, b_ref, o_ref, *, relu):
    acc = jnp.dot(a_ref[...], b_ref[...], preferred_element_type=jnp.float32)
    if relu:
        acc = jnp.maximum(acc, 0.0)
    o_ref[...] = acc


def _mm(a, b, relu=False, bm=512):
    m, k = a.shape
    n = b.shape[1]
    grid = (m // bm,)
    return pl.pallas_call(
        functools.partial(_mm_body, relu=relu),
        grid=grid,
        in_specs=[
            pl.BlockSpec((bm, k), lambda i: (i, 0)),
            pl.BlockSpec((k, n), lambda i: (0, 0)),
        ],
        out_specs=pl.BlockSpec((bm, n), lambda i: (i, 0)),
        out_shape=jax.ShapeDtypeStruct((m, n), jnp.float32),
    )(a, b)


def _edge_head_body(pg_ref, qg_ref, wb_ref, bab_ref, o_ref):
    a = jnp.maximum(pg_ref[...] + qg_ref[...] + bab_ref[...], 0.0)
    o_ref[...] = jax.lax.dot_general(
        a, wb_ref[...], (((1,), (0,)), ((), ())),
        preferred_element_type=jnp.float32)


def _edge_heads(pg, qg, wb_cat, ba_cat, bb_vec):
    e = pg.shape[0]
    be = 4096
    out = pl.pallas_call(
        _edge_head_body,
        grid=(e // be,),
        in_specs=[
            pl.BlockSpec((be, 192), lambda i: (i, 0)),
            pl.BlockSpec((be, 192), lambda i: (i, 0)),
            pl.BlockSpec((192, _INIT_K), lambda i: (0, 0)),
            pl.BlockSpec((1, 192), lambda i: (0, 0)),
        ],
        out_specs=pl.BlockSpec((be, _INIT_K), lambda i: (i, 0)),
        out_shape=jax.ShapeDtypeStruct((e, _INIT_K), jnp.float32),
    )(pg, qg, wb_cat, ba_cat)
    return out + bb_vec[None, :]


def _sample_body(g_ref, l_ref, beta_ref, o_ref):
    o_ref[...] = jax.nn.sigmoid((g_ref[...] + l_ref[...]) / beta_ref[0])


def _sample(gates, logits, beta):
    s, e = gates.shape
    be = 16384
    return pl.pallas_call(
        _sample_body,
        grid=(e // be,),
        in_specs=[
            pl.BlockSpec((s, be), lambda i: (0, i)),
            pl.BlockSpec((s, be), lambda i: (0, i)),
            pl.BlockSpec(memory_space=pltpu.SMEM),
        ],
        out_specs=pl.BlockSpec((s, be), lambda i: (0, i)),
        out_shape=jax.ShapeDtypeStruct((s, e), jnp.float32),
    )(gates, logits, beta)


def kernel(x, edge_index, embed, tmp, origin_pred, Wa, ba, Wb, bb, W1, W2, Wout):
    N = x.shape[0]
    E = edge_index.shape[1]
    e0 = edge_index[0].astype(jnp.int32)
    e1 = edge_index[1].astype(jnp.int32)
    beta = jnp.reshape(tmp, (1,))
    nkey = jax.random.key(1)

    gates = []
    for s in range(_INIT_K + 1):
        u = jax.random.uniform(jax.random.fold_in(nkey, s), (E,),
                               minval=1e-6, maxval=1.0 - 1e-6)
        gates.append(jnp.log(u) - jnp.log(1.0 - u))
    gates = jnp.stack(gates)

    WaL = jnp.concatenate([Wa[i, :512, :] for i in range(_INIT_K)], axis=1)
    WaR = jnp.concatenate([Wa[i, 512:, :] for i in range(_INIT_K)], axis=1)
    PQ = _mm(embed, jnp.concatenate([WaL, WaR], axis=1))
    P = PQ[:, :192]
    Q = PQ[:, 192:384]
    xW1 = _mm(x, W1, bm=512)

    pg = P[e0]
    qg = Q[e1]
    wb_cat = jnp.zeros((192, _INIT_K), jnp.float32)
    for i in range(_INIT_K):
        wb_cat = wb_cat.at[64 * i:64 * (i + 1), i].set(Wb[i][:, 0])
    ba_cat = jnp.concatenate([ba[i] for i in range(_INIT_K)]).reshape(1, 192)
    bb_vec = jnp.concatenate([bb[i] for i in range(_INIT_K)])
    e_h = _edge_heads(pg, qg, wb_cat, ba_cat, bb_vec)
    mask_logits = jnp.max(e_h, axis=1)

    logits_all = jnp.concatenate([mask_logits[None], e_h.T], axis=0)
    vals = _sample(gates, logits_all, beta)

    key = e0 * N + e1
    rkey = e1 * N + e0
    T = jnp.zeros((N * N, 4), jnp.float32).at[key].add(vals.T)
    emask_all = (T[key] + T[rkey]).T * 0.5
    edge_mask = emask_all[0]
    demask = emask_all[1:]

    e0g = e0.reshape(E // _CHUNK, _CHUNK)
    e1g = e1.reshape(E // _CHUNK, _CHUNK)
    z256 = jnp.zeros((256, 128), jnp.float32)

    def run_model(m):
        p = _seg_pass(xW1, e0g, e1g, m, z256)
        h = jnp.maximum(p[0] + p[1], 0.0)
        p2 = _seg_pass(h, e0g, e1g, m, z256)
        h2 = _mm(p2[0] + p2[1], W2, relu=True)
        logits = (jnp.sum(h2, axis=0) / N) @ Wout
        return logits, jax.nn.softmax(logits)

    logits0, _ = run_model(edge_mask)
    res = jax.nn.softmax(logits0)

    k = round(_TRAIN_TOPK / 100.0 * E)

    def kth_largest(v):
        bits = lax.bitcast_convert_type(v, jnp.int32)
        lo = jnp.zeros((3,), jnp.int32)
        hi = jnp.full((3,), 0x7f800000, jnp.int32)

        def body(_, lohi):
            lo, hi = lohi
            mid = lo + (hi - lo + 1) // 2
            cnt = jnp.sum(bits >= mid[:, None], axis=1)
            ok = cnt >= k
            return jnp.where(ok, mid, lo), jnp.where(ok, hi, mid - 1)

        lo, hi = lax.fori_loop(0, 31, body, (lo, hi))
        return lax.bitcast_convert_type(lo, jnp.float32)

    tks = kth_largest(demask)

    def select(v, t):
        g = jnp.sum(v > t)
        eq = (v == t)
        tie = jnp.cumsum(eq.astype(jnp.int32)) - eq.astype(jnp.int32)
        return (v > t) | (eq & (g + tie < k))

    fminus, fplus, sel_last = [], [], None
    for i in range(_INIT_K):
        s = select(demask[i], tks[i])
        sel_last = s
        _, p1 = run_model(jnp.where(s, 1.0, demask[i]))
        fminus.append(jnp.sum(jnp.abs(origin_pred - p1)))
        _, p2 = run_model(jnp.where(s, 1.0 - demask[i], 1.0))
        fplus.append(jnp.sum(jnp.abs(origin_pred - p2)))

    disen_max = jnp.max(demask, axis=0)
    pair = jnp.zeros((_NC,), jnp.float32)
    for i in range(_INIT_K):
        for j in range(i + 1, _INIT_K):
            mnt = jnp.where(sel_last, 1.0, jnp.maximum(demask[i], demask[j]))
            _, pp = run_model(mnt)
            pair = pair + pp

    return (res, demask, disen_max, jnp.stack(fminus), jnp.stack(fplus), pair)
